# baseline (speedup 1.0000x reference)
import functools

import jax
import jax.numpy as jnp
from jax import lax
from jax.experimental import pallas as pl
from jax.experimental.pallas import tpu as pltpu
from jax.experimental.pallas import tpu_sc as plsc

NC = 2
NS = 16
NW = NC * NS
LANES = 16

BATCH = 16384
EMBED_DIM = 32
PER_W = BATCH // NW
BLOCK = 64
NBLOCK = PER_W // BLOCK
GROUPS = BLOCK // LANES


def _mf_kernel(user_ids, show_ids, user_emb, show_emb, user_bias, show_bias,
               out, idx_u, idx_s, ue_t, se_t, ub_t, sb_t, out_v, sem):
    wid = lax.axis_index("c") * NS + lax.axis_index("s")
    base = wid * PER_W

    pltpu.sync_copy(user_ids.at[pl.ds(base, PER_W)], idx_u)
    pltpu.sync_copy(show_ids.at[pl.ds(base, PER_W)], idx_s)

    iota = lax.iota(jnp.int32, LANES)
    zeros = jnp.zeros((LANES,), jnp.int32)
    cols = [(iota + d) % EMBED_DIM for d in range(EMBED_DIM)]

    def block_body(b, _):
        off = b * BLOCK
        for k0 in range(0, BLOCK, LANES):
            uids = idx_u[pl.ds(off + k0, LANES)]
            sids = idx_s[pl.ds(off + k0, LANES)]
            for k in range(LANES):
                uid = uids[k]
                sid = sids[k]
                pltpu.async_copy(user_emb.at[uid], ue_t.at[k0 + k], sem)
                pltpu.async_copy(show_emb.at[sid], se_t.at[k0 + k], sem)
                pltpu.async_copy(user_bias.at[uid], ub_t.at[k0 + k], sem)
                pltpu.async_copy(show_bias.at[sid], sb_t.at[k0 + k], sem)
        pltpu.make_async_copy(user_emb.at[pl.ds(0, BLOCK)], ue_t, sem).wait()
        pltpu.make_async_copy(show_emb.at[pl.ds(0, BLOCK)], se_t, sem).wait()
        pltpu.make_async_copy(user_bias.at[pl.ds(0, BLOCK)], ub_t, sem).wait()
        pltpu.make_async_copy(show_bias.at[pl.ds(0, BLOCK)], sb_t, sem).wait()

        for g in range(GROUPS):
            row = g * LANES + iota
            accs = [jnp.zeros((LANES,), jnp.float32) for _ in range(4)]
            for d in range(EMBED_DIM):
                u = plsc.load_gather(ue_t, [row, cols[d]])
                s = plsc.load_gather(se_t, [row, cols[d]])
                accs[d % 4] = accs[d % 4] + u * s
            ub = plsc.load_gather(ub_t, [row, zeros])
            sb = plsc.load_gather(sb_t, [row, zeros])
            res = (accs[0] + accs[1]) + (accs[2] + accs[3]) + (ub + sb)
            out_v[pl.ds(off + g * LANES, LANES)] = res
        return 0

    lax.fori_loop(0, NBLOCK, block_body, 0)

    pltpu.sync_copy(out_v, out.at[pl.ds(base, PER_W)])


@jax.jit
def _mf(user_ids, show_ids, user_emb, show_emb, user_bias, show_bias):
    mesh = plsc.VectorSubcoreMesh(
        core_axis_name="c", subcore_axis_name="s",
        num_cores=NC, num_subcores=NS)
    fn = pl.kernel(
        _mf_kernel,
        out_type=jax.ShapeDtypeStruct((BATCH,), jnp.float32),
        mesh=mesh,
        scratch_types=[
            pltpu.VMEM((PER_W,), jnp.int32),
            pltpu.VMEM((PER_W,), jnp.int32),
            pltpu.VMEM((BLOCK, EMBED_DIM), jnp.float32),
            pltpu.VMEM((BLOCK, EMBED_DIM), jnp.float32),
            pltpu.VMEM((BLOCK, 1), jnp.float32),
            pltpu.VMEM((BLOCK, 1), jnp.float32),
            pltpu.VMEM((PER_W,), jnp.float32),
            pltpu.SemaphoreType.DMA,
        ],
        compiler_params=pltpu.CompilerParams(
            needs_layout_passes=False, use_tc_tiling_on_sc=True),
    )
    return fn(user_ids, show_ids, user_emb, show_emb, user_bias, show_bias)


def kernel(user_ids, show_ids, user_emb, show_emb, user_bias, show_bias):
    return _mf(user_ids.astype(jnp.int32), show_ids.astype(jnp.int32),
               user_emb, show_emb, user_bias, show_bias)
